# baseline (speedup 1.0000x reference)
import functools

import jax
import jax.numpy as jnp
from jax import lax
from jax.experimental import pallas as pl
from jax.experimental.pallas import tpu as pltpu
from jax.experimental.pallas import tpu_sc as plsc

_NC = 2
_NS = 16
_NW = _NC * _NS
_CHUNK = 128


def _merge_body(scale, main_ref, a_ref, bt_ref, out_ref):
    out_ref[...] = main_ref[...] + scale * jnp.dot(
        a_ref[...], bt_ref[...], preferred_element_type=jnp.float32
    )


def _merged_table(main_weight, lora_a, lora_bt, scale):
    v, d = main_weight.shape
    r = lora_a.shape[1]
    block = 5000
    grid = v // block
    return pl.pallas_call(
        functools.partial(_merge_body, scale),
        grid=(grid,),
        in_specs=[
            pl.BlockSpec((block, d), lambda i: (i, 0)),
            pl.BlockSpec((block, r), lambda i: (i, 0)),
            pl.BlockSpec((r, d), lambda i: (0, 0)),
        ],
        out_specs=pl.BlockSpec((block, d), lambda i: (i, 0)),
        out_shape=jax.ShapeDtypeStruct((v, d), jnp.float32),
    )(main_weight, lora_a, lora_bt)


_NBUF = 5


def _make_gather(nchunk, d):
    n_per_w = nchunk * _CHUNK
    assert nchunk % _NBUF == 0
    ngroups = nchunk // _NBUF
    mesh = plsc.VectorSubcoreMesh(
        core_axis_name="c", subcore_axis_name="s", num_cores=_NC, num_subcores=_NS
    )

    @functools.partial(
        pl.kernel,
        out_type=jax.ShapeDtypeStruct((_NW * n_per_w, d), jnp.float32),
        mesh=mesh,
        scratch_types=[
            pltpu.VMEM((n_per_w,), jnp.int32),
            pltpu.VMEM((_NBUF, _CHUNK, d), jnp.float32),
            [pltpu.SemaphoreType.DMA] * _NBUF,
            [pltpu.SemaphoreType.DMA] * _NBUF,
        ],
    )
    def gather(table_hbm, idx_hbm, out_hbm, idx_v, rows_v, gsem, dsem):
        wid = lax.axis_index("s") * _NC + lax.axis_index("c")
        base = wid * n_per_w
        pltpu.sync_copy(idx_hbm.at[wid], idx_v)

        def fire(j, slot):
            pltpu.async_copy(
                table_hbm.at[idx_v.at[pl.ds(j * _CHUNK, _CHUNK)]],
                rows_v.at[slot],
                gsem[slot],
            )

        def gwait(slot):
            pltpu.make_async_copy(
                table_hbm.at[idx_v.at[pl.ds(0, _CHUNK)]], rows_v.at[slot], gsem[slot]
            ).wait()

        def drain(j, slot):
            pltpu.async_copy(
                rows_v.at[slot], out_hbm.at[pl.ds(base + j * _CHUNK, _CHUNK)], dsem[slot]
            )

        def dwait(slot):
            pltpu.make_async_copy(
                rows_v.at[slot], out_hbm.at[pl.ds(base, _CHUNK)], dsem[slot]
            ).wait()

        for k in range(_NBUF - 1):
            fire(k, k)

        def group(g, carry):
            j0 = g * _NBUF
            for k in range(_NBUF):
                s = k
                s3 = (k + _NBUF - 1) % _NBUF
                gwait(s)
                drain(j0 + k, s)
                jnext = j0 + k + _NBUF - 1

                @pl.when(jnext < nchunk)
                def _():
                    @pl.when(g + k > 0)
                    def _():
                        dwait(s3)

                    fire(jnext, s3)

            return carry

        lax.fori_loop(0, ngroups, group, 0)
        for k in range(_NBUF):
            dwait(k)

    return gather


def kernel(idx, main_weight, lora_A, lora_B):
    b, l = idx.shape
    v, d = main_weight.shape
    rank = lora_A.shape[1]
    alpha = 32.0
    scale = alpha / rank

    merged = _merged_table(main_weight, lora_A, lora_B.T, scale)

    n = b * l
    assert n % (_NW * _CHUNK) == 0
    nchunk = n // (_NW * _CHUNK)
    idx2 = idx.astype(jnp.int32).T.reshape(_NW, nchunk * _CHUNK)
    rows = _make_gather(nchunk, d)(merged, idx2)
    return rows.reshape(l, b, d).transpose(1, 0, 2)
